# baseline (speedup 1.0000x reference)
import jax
import jax.numpy as jnp
from jax.experimental import pallas as pl
from jax.experimental.pallas import tpu as pltpu

_N_CHUNKS = 16


def _copy_body(t_hbm, o_hbm, vmem, sems_in, sem_out):
    length = vmem.shape[0]
    batch = o_hbm.shape[0]
    ch = length // _N_CHUNKS
    for c in range(_N_CHUNKS):
        pltpu.make_async_copy(
            t_hbm.at[pl.ds(c * ch, ch)], vmem.at[pl.ds(c * ch, ch)], sems_in.at[c]
        ).start()
    for c in range(_N_CHUNKS):
        pltpu.make_async_copy(
            t_hbm.at[pl.ds(c * ch, ch)], vmem.at[pl.ds(c * ch, ch)], sems_in.at[c]
        ).wait()
        for b in range(batch):
            pltpu.make_async_copy(
                vmem.at[pl.ds(c * ch, ch)], o_hbm.at[b, pl.ds(c * ch, ch)], sem_out
            ).start()
    for c in range(_N_CHUNKS):
        for b in range(batch):
            pltpu.make_async_copy(
                vmem.at[pl.ds(c * ch, ch)], o_hbm.at[b, pl.ds(c * ch, ch)], sem_out
            ).wait()


def kernel(x, table):
    batch, length = x.shape
    feat = table.shape[1]
    return pl.pallas_call(
        _copy_body,
        in_specs=[pl.BlockSpec(memory_space=pl.ANY)],
        out_specs=pl.BlockSpec(memory_space=pl.ANY),
        out_shape=jax.ShapeDtypeStruct((batch, length, feat), table.dtype),
        scratch_shapes=[
            pltpu.VMEM((length, feat), table.dtype),
            pltpu.SemaphoreType.DMA((_N_CHUNKS,)),
            pltpu.SemaphoreType.DMA,
        ],
    )(table)


# speedup vs baseline: 1.0174x; 1.0174x over previous
import jax
import jax.numpy as jnp
from jax.experimental import pallas as pl
from jax.experimental.pallas import tpu as pltpu

_N_CHUNKS = 4


def _copy_body(t_hbm, o_hbm, vmem, sems_in, sem_out):
    length = vmem.shape[0]
    batch = o_hbm.shape[0]
    ch = length // _N_CHUNKS
    for c in range(_N_CHUNKS):
        pltpu.make_async_copy(
            t_hbm.at[pl.ds(c * ch, ch)], vmem.at[pl.ds(c * ch, ch)], sems_in.at[c]
        ).start()
    for c in range(_N_CHUNKS):
        pltpu.make_async_copy(
            t_hbm.at[pl.ds(c * ch, ch)], vmem.at[pl.ds(c * ch, ch)], sems_in.at[c]
        ).wait()
        for b in range(batch):
            pltpu.make_async_copy(
                vmem.at[pl.ds(c * ch, ch)], o_hbm.at[b, pl.ds(c * ch, ch)], sem_out
            ).start()
    for c in range(_N_CHUNKS):
        for b in range(batch):
            pltpu.make_async_copy(
                vmem.at[pl.ds(c * ch, ch)], o_hbm.at[b, pl.ds(c * ch, ch)], sem_out
            ).wait()


def kernel(x, table):
    batch, length = x.shape
    feat = table.shape[1]
    return pl.pallas_call(
        _copy_body,
        in_specs=[pl.BlockSpec(memory_space=pl.ANY)],
        out_specs=pl.BlockSpec(memory_space=pl.ANY),
        out_shape=jax.ShapeDtypeStruct((batch, length, feat), table.dtype),
        scratch_shapes=[
            pltpu.VMEM((length, feat), table.dtype),
            pltpu.SemaphoreType.DMA((_N_CHUNKS,)),
            pltpu.SemaphoreType.DMA,
        ],
    )(table)
